# baseline (speedup 1.0000x reference)
import functools

import jax
import jax.numpy as jnp
from jax import lax
from jax.experimental import pallas as pl
from jax.experimental.pallas import tpu as pltpu

_EMPHASIS = 1.5
_TOPK_FRAC = 0.2


def _attn_body(nb, k_value, Wt_ref, b_ref, x_ref, sum_ref, emph_ref, xbf_ref):
    int_min = jnp.int32(-(2 ** 31))
    T = x_ref.shape[1]

    xbf_ref[...] = x_ref[...].astype(jnp.bfloat16)
    Wt = Wt_ref[...]

    rows = [
        lax.dot_general(
            Wt, x_ref[b], dimension_numbers=(((1,), (1,)), ((), ())),
            preferred_element_type=jnp.float32)
        for b in range(nb)
    ]
    pre = jnp.concatenate(rows, axis=0)[:, None, :]
    e = jnp.tanh(pre + b_ref[0, 0])
    m = jnp.max(e, axis=2, keepdims=True)
    ex = jnp.exp(e - m)
    z = jnp.sum(ex, axis=2, keepdims=True)
    a = ex / z

    bits = lax.bitcast_convert_type(a, jnp.int32)
    skey = jnp.where(bits >= 0, bits, bits ^ jnp.int32(0x7FFFFFFF))
    skey = jnp.where(bits == int_min, jnp.int32(0), skey)

    jv15 = lax.broadcasted_iota(jnp.int32, (nb, 15, 1), 1) + 1
    p_u = jnp.zeros((nb, 1, 1), jnp.int32)
    for rnd in range(8):
        shift = 28 - 4 * rnd
        cand = p_u | lax.shift_left(jv15, shift)
        scand = cand ^ int_min
        cmp = (skey >= scand).astype(jnp.int32)
        c = jnp.sum(cmp, axis=2, keepdims=True)
        j_star = jnp.sum((c >= k_value).astype(jnp.int32),
                         axis=1, keepdims=True)
        p_u = p_u | lax.shift_left(j_star, shift)
    s_star = p_u ^ int_min

    gt = skey > s_star
    g = jnp.sum(gt.astype(jnp.int32), axis=2, keepdims=True)
    r = k_value - g
    eq = skey == s_star
    idx = lax.broadcasted_iota(jnp.int32, (nb, 1, T), 2)
    jv16 = lax.broadcasted_iota(jnp.int32, (nb, 16, 1), 1)
    p_i = jnp.zeros((nb, 1, 1), jnp.int32)
    for sh in (8, 4, 0):
        low = (1 << sh) - 1
        t_test = p_i | lax.shift_left(jv16, sh) | low
        hit = (eq & (idx <= t_test)).astype(jnp.int32)
        f = jnp.sum(hit, axis=2, keepdims=True)
        n_star = jnp.sum((f < r).astype(jnp.int32),
                         axis=1, keepdims=True)
        p_i = p_i | lax.shift_left(n_star, sh)
    mask = gt | (eq & (idx <= p_i))

    w = jnp.where(mask, a * jnp.float32(_EMPHASIS), a)
    emph_ref[...] = w
    wbf = w.astype(jnp.bfloat16)
    for b in range(nb):
        sum_ref[b] = lax.dot_general(
            wbf[b], xbf_ref[b], dimension_numbers=(((1,), (0,)), ((), ())),
            preferred_element_type=jnp.float32)


@jax.jit
def kernel(x, W, b):
    B, T, D = x.shape
    nb = 2
    k_value = max(int(T * _TOPK_FRAC), 1)
    Wt = W.reshape(1, D)
    b2 = b.reshape(1, 1)

    body = functools.partial(_attn_body, nb, k_value)
    summed, emph = pl.pallas_call(
        body,
        grid=(B // nb,),
        in_specs=[
            pl.BlockSpec((1, D), lambda i: (0, 0)),
            pl.BlockSpec((1, 1), lambda i: (0, 0)),
            pl.BlockSpec((nb, T, D), lambda i: (i, 0, 0)),
        ],
        out_specs=[
            pl.BlockSpec((nb, 1, D), lambda i: (i, 0, 0)),
            pl.BlockSpec((nb, 1, T), lambda i: (i, 0, 0)),
        ],
        out_shape=[
            jax.ShapeDtypeStruct((B, 1, D), jnp.float32),
            jax.ShapeDtypeStruct((B, 1, T), jnp.float32),
        ],
        scratch_shapes=[pltpu.VMEM((nb, T, D), jnp.bfloat16)],
        compiler_params=pltpu.CompilerParams(
            vmem_limit_bytes=100 * 1024 * 1024,
        ),
    )(Wt, b2, x)
    return summed.reshape(B, D), emph.reshape(B, T)
